# baseline (speedup 1.0000x reference)
import functools

import jax
import jax.numpy as jnp
from jax.experimental import pallas as pl
from jax.experimental.pallas import tpu as pltpu


def _se_slab_body(x_ref, onesw_ref, w1t_ref, w2t_ref, o_any, wsem, *, k):
    g = pl.program_id(0)

    def _compute_one(i, _):
        x = x_ref[i]
        pooled = jax.lax.dot_general(
            x, onesw_ref[...], (((1,), (0,)), ((), ())),
            preferred_element_type=jnp.float32)
        hidden = jnp.maximum(
            jax.lax.dot_general(w1t_ref[...], pooled,
                                (((1,), (0,)), ((), ())),
                                preferred_element_type=jnp.float32), 0.0)
        gate = jax.nn.sigmoid(
            jax.lax.dot_general(w2t_ref[...], hidden,
                                (((1,), (0,)), ((), ())),
                                preferred_element_type=jnp.float32))
        x_ref[i] = x * gate[:, :1]
        return ()

    jax.lax.fori_loop(0, k, _compute_one, ())

    copy = pltpu.make_async_copy(x_ref, o_any.at[pl.ds(g * k, k)], wsem)
    copy.start()
    copy.wait()


def kernel(x_nchw, w1, w2):
    B, C, H, W = x_nchw.shape
    Cr = w1.shape[1]
    HW = H * W
    x_flat = x_nchw.reshape(B, C, HW)

    k = 8 if B % 8 == 0 else (4 if B % 4 == 0 else (2 if B % 2 == 0 else 1))
    n_groups = B // k

    out_flat = pl.pallas_call(
        functools.partial(_se_slab_body, k=k),
        out_shape=jax.ShapeDtypeStruct((B, C, HW), x_nchw.dtype),
        grid=(n_groups,),
        in_specs=[
            pl.BlockSpec((k, C, HW), lambda g: (g, 0, 0)),
            pl.BlockSpec((HW, 128), lambda g: (0, 0)),
            pl.BlockSpec((Cr, C), lambda g: (0, 0)),
            pl.BlockSpec((C, Cr), lambda g: (0, 0)),
        ],
        out_specs=pl.BlockSpec(memory_space=pl.ANY),
        scratch_shapes=[pltpu.SemaphoreType.DMA],
        compiler_params=pltpu.CompilerParams(
            dimension_semantics=("arbitrary",),
            vmem_limit_bytes=58 * 1024 * 1024),
    )(x_flat, jnp.full((HW, 128), 1.0 / float(HW), jnp.float32), w1.T, w2.T)
    return out_flat.reshape(B, C, H, W)


# speedup vs baseline: 1.3206x; 1.3206x over previous
import jax
import jax.numpy as jnp
from jax.experimental import pallas as pl
from jax.experimental.pallas import tpu as pltpu


def _se_gate_body(x_ref, onesw_ref, w1t_ref, w2t_ref, g_ref):
    pooled = jax.lax.dot_general(
        x_ref[0], onesw_ref[...], (((1,), (0,)), ((), ())),
        preferred_element_type=jnp.float32)
    hidden = jnp.maximum(
        jax.lax.dot_general(w1t_ref[...], pooled,
                            (((1,), (0,)), ((), ())),
                            preferred_element_type=jnp.float32), 0.0)
    gate = jax.nn.sigmoid(
        jax.lax.dot_general(w2t_ref[...], hidden,
                            (((1,), (0,)), ((), ())),
                            preferred_element_type=jnp.float32))
    g_ref[0] = gate


def kernel(x_nchw, w1, w2):
    B, C, H, W = x_nchw.shape
    Cr = w1.shape[1]
    HW = H * W
    x_flat = x_nchw.reshape(B, C, HW)

    gates = pl.pallas_call(
        _se_gate_body,
        out_shape=jax.ShapeDtypeStruct((B, C, 128), jnp.float32),
        grid=(B,),
        in_specs=[
            pl.BlockSpec((1, C, HW), lambda b: (b, 0, 0)),
            pl.BlockSpec((HW, 128), lambda b: (0, 0)),
            pl.BlockSpec((Cr, C), lambda b: (0, 0)),
            pl.BlockSpec((C, Cr), lambda b: (0, 0)),
        ],
        out_specs=pl.BlockSpec((1, C, 128), lambda b: (b, 0, 0)),
        compiler_params=pltpu.CompilerParams(
            dimension_semantics=("arbitrary",),
            vmem_limit_bytes=40 * 1024 * 1024),
    )(x_flat, jnp.full((HW, 128), 1.0 / float(HW), jnp.float32), w1.T, w2.T)

    out_flat = x_flat * gates[:, :, :1]
    return out_flat.reshape(B, C, H, W)
